# Initial kernel scaffold; baseline (speedup 1.0000x reference)
import jax
import jax.numpy as jnp
from jax.experimental import pallas as pl


def kernel(x, edge_index, batch, Wl, bl, Wr, br, att, bias, Wlin, blin):
    raise NotImplementedError("write your pallas kernel here")


# speedup vs baseline: 2.1850x; 2.1850x over previous
import jax
import jax.numpy as jnp
from jax.experimental import pallas as pl
from jax.experimental.pallas import tpu as pltpu

N = 10000
NUM_GRAPHS = 64
NEG_SLOPE = 0.2


def _head_body(h_ref, batch_ref, wlin_ref, blin_ref, out_ref):
    h = h_ref[...]
    b = batch_ref[...]
    g = jax.lax.broadcasted_iota(jnp.int32, (NUM_GRAPHS, h.shape[0]), 0)
    m = (b == g).astype(jnp.float32)
    sums = jnp.dot(m, h, preferred_element_type=jnp.float32)
    counts = jnp.sum(m, axis=1, keepdims=True)
    pooled = sums / jnp.clip(counts, 1.0, None)
    logits = jnp.dot(pooled, wlin_ref[...], preferred_element_type=jnp.float32)
    logits = logits + blin_ref[...]
    mx = jnp.max(logits, axis=1, keepdims=True)
    z = logits - mx
    lse = jnp.log(jnp.sum(jnp.exp(z), axis=1, keepdims=True))
    out_ref[...] = z - lse


def _head(h, batch, Wlin, blin):
    return pl.pallas_call(
        _head_body,
        out_shape=jax.ShapeDtypeStruct((NUM_GRAPHS, Wlin.shape[1]), jnp.float32),
    )(h, batch.reshape(1, -1), Wlin, blin.reshape(1, -1))


def kernel(x, edge_index, batch, Wl, bl, Wr, br, att, bias, Wlin, blin):
    n = x.shape[0]
    loops = jnp.arange(n, dtype=edge_index.dtype)
    src = jnp.concatenate([edge_index[0], loops])
    dst = jnp.concatenate([edge_index[1], loops])
    xl = x @ Wl + bl
    xr = x @ Wr + br
    feat = jax.nn.leaky_relu(xl[src] + xr[dst], NEG_SLOPE)
    e = feat @ att
    ex = jnp.exp(e)
    denom = jax.ops.segment_sum(ex, dst, num_segments=n)
    numer = jax.ops.segment_sum(ex[:, None] * xl[src], dst, num_segments=n)
    out = numer / (denom[:, None] + 1e-16)
    h = jax.nn.relu(out + bias)
    return _head(h, batch, Wlin, blin)


# speedup vs baseline: 7.7968x; 3.5684x over previous
import jax
import jax.numpy as jnp
from jax import lax
from jax.experimental import pallas as pl
from jax.experimental.pallas import tpu as pltpu
from jax.experimental.pallas import tpu_sc as plsc

N = 10000
E = 320000
HID = 128
OUT_CH = 10
NUM_GRAPHS = 64
NEG_SLOPE = 0.2

NPAD = 10240
DUMMY = N
K = 128
NW = 32
EDGES = E + N
CHUNKS = -(-EDGES // (NW * K))
EP = NW * K * CHUNKS
PW = K * CHUNKS
HALF = NPAD // 2
ACC_ROWS = 5376
RPT = ACC_ROWS // 16
HPT = HALF // 16


def _xform_body(x_ref, wl_ref, bl_ref, wr_ref, br_ref, xl_ref, xr_ref):
    x = x_ref[...]
    xl_ref[...] = jnp.dot(x, wl_ref[...], preferred_element_type=jnp.float32) + bl_ref[...]
    xr_ref[...] = jnp.dot(x, wr_ref[...], preferred_element_type=jnp.float32) + br_ref[...]


def _sc_edge_kernel(xl_hbm, xr_hbm, src_hbm, dst_hbm, att_hbm,
                    numer_hbm, denom_hbm,
                    idx_src, idx_dst, idx_loc, xl_rows, xr_rows, out_rows,
                    accbuf, kbuf, att_v, wstore, denom_t, acc_sh,
                    sem_l, sem_r):
    c = lax.axis_index("c")
    s = lax.axis_index("s")
    wid = s * 2 + c

    pltpu.sync_copy(att_hbm, att_v)
    att_vecs = [att_v[pl.ds(j * 16, 16)] for j in range(8)]
    iota = lax.iota(jnp.int32, 16)
    spread = HALF + (iota & 7)
    zeros16 = jnp.zeros((16,), jnp.float32)

    def _zero_den(i, _):
        denom_t[pl.ds(i * 16, 16)] = zeros16
        return 0
    lax.fori_loop(0, NPAD // 16, _zero_den, 0)

    def _zero_acc():
        def _zero_row(i, _):
            for j in range(HID // 16):
                out_rows[i, pl.ds(j * 16, 16)] = zeros16
            return 0
        lax.fori_loop(0, K, _zero_row, 0)
        for r0 in (0, 128, 256):
            n = min(128, RPT - r0)
            pltpu.sync_copy(out_rows.at[pl.ds(0, n)],
                            acc_sh.at[pl.ds(s * RPT + r0, n)])

    def _dump_half(row_base):
        for r0 in (0, 128, 256):
            n = min(128, HPT - r0)
            r = s * HPT + r0
            pltpu.sync_copy(acc_sh.at[pl.ds(r, n)], out_rows.at[pl.ds(0, n)])
            pltpu.sync_copy(out_rows.at[pl.ds(0, n)],
                            numer_hbm.at[c, pl.ds(row_base + r, n)])

    _zero_acc()
    plsc.subcore_barrier()

    def chunk_body1(ci, carry):
        base = wid * PW + ci * K
        pltpu.sync_copy(src_hbm.at[pl.ds(base, K)], idx_src)
        pltpu.sync_copy(dst_hbm.at[pl.ds(base, K)], idx_dst)
        cp_l = pltpu.async_copy(xl_hbm.at[idx_src], xl_rows, sem_l)
        cp_r = pltpu.async_copy(xr_hbm.at[idx_dst], xr_rows, sem_r)
        cp_l.wait()
        cp_r.wait()

        def group(g, _):
            b = pl.multiple_of(g * 16, 16)
            for e in range(16):
                acc = zeros16
                for j in range(8):
                    u = (xl_rows[b + e, pl.ds(j * 16, 16)]
                         + xr_rows[b + e, pl.ds(j * 16, 16)])
                    lr = jnp.where(u >= 0.0, u, u * NEG_SLOPE)
                    acc = acc + att_vecs[j] * lr
                accbuf[pl.ds(e * 16, 16)] = acc
            esum = zeros16
            for d in range(16):
                esum = esum + plsc.load_gather(accbuf, [iota * 16 + d])
            w = jnp.exp(esum)
            wstore[pl.ds(ci * K + b, 16)] = w
            for e in range(16):
                ws = w[e]
                for j in range(HID // 16):
                    out_rows[b + e, pl.ds(j * 16, 16)] = (
                        xl_rows[b + e, pl.ds(j * 16, 16)] * ws)
            dst16 = idx_dst[pl.ds(b, 16)]
            idx_loc[pl.ds(b, 16)] = jnp.where(dst16 < HALF, dst16, spread)
            kd, vw = plsc.sort_key_val(dst16, w)
            cum = plsc.cumsum(vw)
            kbuf[...] = kd
            knext = plsc.load_gather(kbuf, [jnp.minimum(iota + 1, 15)])
            last = (kd != knext) | (iota == 15)
            plsc.addupdate_scatter(denom_t, [kd], cum, mask=last)
            plsc.addupdate_scatter(denom_t, [knext], -cum,
                                   mask=last & (iota < 15))
            return 0

        lax.fori_loop(0, K // 16, group, 0)
        pltpu.sync_copy(out_rows, acc_sh.at[idx_loc], add=True)
        return carry

    lax.fori_loop(0, CHUNKS, chunk_body1, 0)
    plsc.subcore_barrier()
    _dump_half(0)
    plsc.subcore_barrier()
    _zero_acc()
    plsc.subcore_barrier()

    def chunk_body2(ci, carry):
        base = wid * PW + ci * K
        pltpu.sync_copy(src_hbm.at[pl.ds(base, K)], idx_src)
        pltpu.sync_copy(dst_hbm.at[pl.ds(base, K)], idx_dst)
        cp_l = pltpu.async_copy(xl_hbm.at[idx_src], xl_rows, sem_l)
        cp_l.wait()

        def group(g, _):
            b = pl.multiple_of(g * 16, 16)
            w = wstore[pl.ds(ci * K + b, 16)]
            for e in range(16):
                ws = w[e]
                for j in range(HID // 16):
                    out_rows[b + e, pl.ds(j * 16, 16)] = (
                        xl_rows[b + e, pl.ds(j * 16, 16)] * ws)
            dst16 = idx_dst[pl.ds(b, 16)]
            idx_loc[pl.ds(b, 16)] = jnp.where(dst16 >= HALF, dst16 - HALF,
                                              spread)
            return 0

        lax.fori_loop(0, K // 16, group, 0)
        pltpu.sync_copy(out_rows, acc_sh.at[idx_loc], add=True)
        return carry

    lax.fori_loop(0, CHUNKS, chunk_body2, 0)
    plsc.subcore_barrier()
    _dump_half(HALF)
    pltpu.sync_copy(denom_t, denom_hbm.at[wid])


def _sc_edge_call(xl, xr, src, dst, att):
    mesh = plsc.VectorSubcoreMesh(core_axis_name="c", subcore_axis_name="s")
    return pl.kernel(
        _sc_edge_kernel,
        out_type=(jax.ShapeDtypeStruct((2, NPAD, HID), jnp.float32),
                  jax.ShapeDtypeStruct((NW, NPAD), jnp.float32)),
        mesh=mesh,
        compiler_params=pltpu.CompilerParams(needs_layout_passes=False),
        scratch_types=[
            pltpu.VMEM((K,), jnp.int32),
            pltpu.VMEM((K,), jnp.int32),
            pltpu.VMEM((K,), jnp.int32),
            pltpu.VMEM((K, HID), jnp.float32),
            pltpu.VMEM((K, HID), jnp.float32),
            pltpu.VMEM((K, HID), jnp.float32),
            pltpu.VMEM((256,), jnp.float32),
            pltpu.VMEM((16,), jnp.int32),
            pltpu.VMEM((HID,), jnp.float32),
            pltpu.VMEM((PW,), jnp.float32),
            pltpu.VMEM((NPAD,), jnp.float32),
            pltpu.VMEM_SHARED((ACC_ROWS, HID), jnp.float32),
            pltpu.SemaphoreType.DMA,
            pltpu.SemaphoreType.DMA,
        ],
    )(xl, xr, src, dst, att)


def _final_body(numer_ref, denom_ref, bias_ref, batch_ref, wlin_ref, blin_ref,
                out_ref):
    numer = numer_ref[0] + numer_ref[1]
    denom = jnp.sum(denom_ref[...], axis=0)[:, None]
    h = jnp.maximum(numer / (denom + 1e-16) + bias_ref[...], 0.0)
    b = batch_ref[...]
    g = lax.broadcasted_iota(jnp.int32, (NUM_GRAPHS, NPAD), 0)
    m = (b == g).astype(jnp.float32)
    sums = jnp.dot(m, h, preferred_element_type=jnp.float32)
    counts = jnp.sum(m, axis=1, keepdims=True)
    pooled = sums / jnp.clip(counts, 1.0, None)
    logits = jnp.dot(pooled, wlin_ref[...], preferred_element_type=jnp.float32)
    logits = logits + blin_ref[...]
    mx = jnp.max(logits, axis=1, keepdims=True)
    z = logits - mx
    out_ref[...] = z - jnp.log(jnp.sum(jnp.exp(z), axis=1, keepdims=True))


def kernel(x, edge_index, batch, Wl, bl, Wr, br, att, bias, Wlin, blin):
    loops = jnp.arange(N, dtype=edge_index.dtype)
    pad = jnp.full((EP - EDGES,), DUMMY, dtype=edge_index.dtype)
    src = jnp.concatenate([edge_index[0], loops, pad])
    dst = jnp.concatenate([edge_index[1], loops, pad])
    x_pad = jnp.pad(x, ((0, NPAD - N), (0, 0)))

    xl, xr = pl.pallas_call(
        _xform_body,
        out_shape=(jax.ShapeDtypeStruct((NPAD, HID), jnp.float32),
                   jax.ShapeDtypeStruct((NPAD, HID), jnp.float32)),
    )(x_pad, Wl, bl.reshape(1, -1), Wr, br.reshape(1, -1))

    numer, denom = _sc_edge_call(xl, xr, src, dst, att)

    batch_pad = jnp.pad(batch, (0, NPAD - N), constant_values=NUM_GRAPHS)
    return pl.pallas_call(
        _final_body,
        out_shape=jax.ShapeDtypeStruct((NUM_GRAPHS, OUT_CH), jnp.float32),
    )(numer, denom, bias.reshape(1, -1), batch_pad.reshape(1, -1),
      Wlin, blin.reshape(1, -1))
